# baseline (speedup 1.0000x reference)
import functools

import jax
import jax.numpy as jnp
from jax import lax
from jax.experimental import pallas as pl
from jax.experimental.pallas import tpu as pltpu
from jax.experimental.pallas import tpu_sc as plsc

_LANES = 16
_NUM_WORKERS = 32
_RB = 8
_NBUF = 2


def _body(n_rows, in_cols, out_cols, s,
          arr_hbm, out_hbm, inb0, inb1, outb0, outb1, si0, si1, so0, so1):
    c = lax.axis_index("c")
    sub = lax.axis_index("s")
    wid = sub * 2 + c
    rows_per_w = n_rows // _NUM_WORKERS
    base_row = wid * rows_per_w
    n_batches = rows_per_w // _RB
    inbs = (inb0, inb1)
    outbs = (outb0, outb1)
    in_sems = (si0, si1)
    out_sems = (so0, so1)
    iota = lax.iota(jnp.int32, _LANES)
    row_idx = [iota * 0 + r for r in range(_RB)]
    g_per_seg = s // _LANES

    def in_cp(i, slot):
        row = base_row + i * _RB
        return pltpu.make_async_copy(
            arr_hbm.at[pl.ds(row, _RB)], inbs[slot], in_sems[slot])

    def out_cp(i, slot):
        row = base_row + i * _RB
        return pltpu.make_async_copy(
            outbs[slot], out_hbm.at[pl.ds(row, _RB)], out_sems[slot])

    def compact(slot):
        src = inbs[slot]
        dst = outbs[slot]

        @plsc.parallel_loop(0, out_cols // _LANES, unroll=8)
        def _(g):
            col = iota + (_LANES * g + g // g_per_seg + 1)
            for r in range(_RB):
                x = plsc.load_gather(src, [row_idx[r], col])
                dst[r, pl.ds(_LANES * g, _LANES)] = x

    for slot in range(_NBUF):
        in_cp(slot, slot).start()

    def step(k, carry):
        for slot in range(_NBUF):
            i = _NBUF * k + slot
            in_cp(i, slot).wait()

            @pl.when(k >= 1)
            def _():
                out_cp(i - _NBUF, slot).wait()

            compact(slot)
            out_cp(i, slot).start()

            @pl.when(k <= n_batches // _NBUF - 2)
            def _():
                in_cp(i + _NBUF, slot).start()
        return carry

    lax.fori_loop(0, n_batches // _NBUF, step, 0)

    for slot in range(_NBUF):
        out_cp(n_batches - _NBUF + slot, slot).wait()


def kernel(arr, S):
    del S
    n_rows, in_cols = arr.shape
    s = int(round(in_cols ** 0.5))
    out_cols = s * (s - 1)
    assert s % _LANES == 0
    assert n_rows % (_NUM_WORKERS * _RB * _NBUF) == 0

    mesh = plsc.VectorSubcoreMesh(core_axis_name="c", subcore_axis_name="s")
    f = pl.kernel(
        functools.partial(_body, n_rows, in_cols, out_cols, s),
        out_type=jax.ShapeDtypeStruct((n_rows, out_cols), jnp.float32),
        mesh=mesh,
        scratch_types=[
            pltpu.VMEM((_RB, in_cols), jnp.float32),
            pltpu.VMEM((_RB, in_cols), jnp.float32),
            pltpu.VMEM((_RB, out_cols), jnp.float32),
            pltpu.VMEM((_RB, out_cols), jnp.float32),
            pltpu.SemaphoreType.DMA,
            pltpu.SemaphoreType.DMA,
            pltpu.SemaphoreType.DMA,
            pltpu.SemaphoreType.DMA,
        ],
        compiler_params=pltpu.CompilerParams(needs_layout_passes=False,
                                             use_tc_tiling_on_sc=True),
    )
    return f(arr)
